# baseline (speedup 1.0000x reference)
import math

import jax
import jax.numpy as jnp
import numpy as np
from jax import lax
from jax.experimental import pallas as pl
from jax.experimental.pallas import tpu as pltpu
from jax.experimental.pallas import tpu_sc as plsc

B = 16
H = 8
D = 128
BS = 16
CTX = 1024
NUM_BLOCKS = 1024
MAXB = 64
ROPE_BASE = 10000.0
HALF = D // 2
SCALE = 1.0 / math.sqrt(D)

HG = 8
HPG = H // HG
ROW = HPG * D
G = 64
NCHK = MAXB * BS // G
NPOS = 2048
TC_BATCHES = 12
SC_BATCHES = B - TC_BATCHES

_invf = ROPE_BASE ** (-np.arange(HALF) / HALF)
_ang = np.arange(NPOS)[:, None] * _invf[None, :]
_PTAB = np.concatenate([np.cos(_ang), np.sin(_ang)], axis=1).astype(np.float32)


_GDN = lax.GatherDimensionNumbers(offset_dims=(), collapsed_slice_dims=(0,),
                                  start_index_map=(0,))


def _iota16():
  return lax.broadcasted_iota(jnp.int32, (16,), 0)


def _dyngather(vec, idxv):
  return lax.gather(vec, idxv[:, None], _GDN, (1,),
                    mode=lax.GatherScatterMode.PROMISE_IN_BOUNDS)


def _allsum(v):
  for sh in (1, 2, 4, 8):
    v = v + _dyngather(v, (_iota16() + sh) & 15)
  return v


def _lanesplat(vec, t):
  return _dyngather(vec, jnp.zeros((16,), jnp.int32) + t)


def _sc_body(q_h, k_h, v_h, kc_h, vc_h, bt_h, sl_h, tab_h, out_h,
             slv, btv, idxr, pidxr, kbuf, vbuf, cbuf, qcs,
             qv, kcv, vcv, outv, ksem, vsem, csem, qsem):
  cid = lax.axis_index("c")
  sid = lax.axis_index("s")
  wid = sid * 2 + cid
  i = wid // HG
  ig = i + TC_BATCHES
  hg = wid % HG
  lane = lax.broadcasted_iota(jnp.int32, (16,), 0)

  pltpu.sync_copy(sl_h, slv)
  pltpu.sync_copy(bt_h.at[ig], btv)
  pltpu.sync_copy(q_h.at[ig, pl.ds(hg * ROW, ROW)], qv)
  pltpu.sync_copy(k_h.at[ig, pl.ds(hg * ROW, ROW)], kcv)
  pltpu.sync_copy(v_h.at[ig, pl.ds(hg * ROW, ROW)], vcv)

  sl_v = _dyngather(slv[...], jnp.full((16,), ig, jnp.int32))
  s_v = 257 + sl_v % (2048 - 257)
  np_v = s_v - 1
  rem_v = np_v % BS
  within_v = np_v < CTX
  nv_v = jnp.where(within_v, np_v, (CTX // BS - 1) * BS + rem_v)
  within_n = (np_v + (lane >> 4)) < CTX
  cur_v = jnp.minimum(np_v, CTX - 1)

  for bb in range(MAXB // 16):
    btq = btv[pl.ds(bb * 16, 16)]
    for b16 in range(16):
      b = bb * 16 + b16
      bt_v = _lanesplat(btq, b16)
      j = b * BS + lane
      rowi = (bt_v * BS + lane) * HG + hg
      posj = jnp.where(within_n, j,
                       jnp.where(j < BS, j, j + BS - 1 - rem_v))
      idxr[pl.ds(b * BS, BS)] = rowi
      pidxr[pl.ds(b * BS, BS)] = posj

  pltpu.async_copy(tab_h.at[cur_v], qcs,
                   qsem).wait()
  q1r, q2r, scur = [], [], []
  for h in range(HPG):
    acc = None
    r1l, r2l = [], []
    for j in range(4):
      c_ = qcs[0, pl.ds(j * 16, 16)]
      s_ = qcs[0, pl.ds(HALF + j * 16, 16)]
      qx1 = qv[pl.ds(h * D + j * 16, 16)]
      qx2 = qv[pl.ds(h * D + HALF + j * 16, 16)]
      kx1 = kcv[pl.ds(h * D + j * 16, 16)]
      kx2 = kcv[pl.ds(h * D + HALF + j * 16, 16)]
      q1 = qx1 * c_ - qx2 * s_
      q2 = qx2 * c_ + qx1 * s_
      k1 = kx1 * c_ - kx2 * s_
      k2 = kx2 * c_ + kx1 * s_
      r1l.append(q1)
      r2l.append(q2)
      t_ = q1 * k1 + q2 * k2
      acc = t_ if acc is None else acc + t_
    q1r.append(r1l)
    q2r.append(r2l)
    scur.append(_allsum(acc) * SCALE)

  def issue(c):
    slot = c % 2
    pltpu.make_async_copy(kc_h.at[idxr.at[pl.ds(c * G, G)]],
                          kbuf.at[slot], ksem.at[slot]).start()
    pltpu.make_async_copy(vc_h.at[idxr.at[pl.ds(c * G, G)]],
                          vbuf.at[slot], vsem.at[slot]).start()
    pltpu.make_async_copy(tab_h.at[pidxr.at[pl.ds(c * G, G)]],
                          cbuf.at[slot], csem.at[slot]).start()

  def wait(c):
    slot = c % 2
    pltpu.make_async_copy(kc_h.at[idxr.at[pl.ds(c * G, G)]],
                          kbuf.at[slot], ksem.at[slot]).wait()
    pltpu.make_async_copy(vc_h.at[idxr.at[pl.ds(c * G, G)]],
                          vbuf.at[slot], vsem.at[slot]).wait()
    pltpu.make_async_copy(tab_h.at[pidxr.at[pl.ds(c * G, G)]],
                          cbuf.at[slot], csem.at[slot]).wait()

  issue(0)

  def chunk_kv(c, carry):
    lvec, acc = carry

    @pl.when(c + 1 < NCHK)
    def _():
      issue(c + 1)
    wait(c)
    slot = c % 2

    def group(g, carry_g):
      lv, a = carry_g
      a = list(a)
      svecs = [jnp.zeros((16,), jnp.float32) for _ in range(HPG)]
      for t in range(16):
        tok = g * 16 + t
        cvs = [cbuf[slot, tok, pl.ds(j * 16, 16)] for j in range(4)]
        svs = [cbuf[slot, tok, pl.ds(HALF + j * 16, 16)] for j in range(4)]
        for h in range(HPG):
          sacc = None
          for j in range(4):
            x1 = kbuf[slot, tok, pl.ds(h * D + j * 16, 16)]
            x2 = kbuf[slot, tok, pl.ds(h * D + HALF + j * 16, 16)]
            r1 = x1 * cvs[j] - x2 * svs[j]
            r2 = x2 * cvs[j] + x1 * svs[j]
            t_ = r1 * q1r[h][j] + r2 * q2r[h][j]
            sacc = t_ if sacc is None else sacc + t_
          s_h = _allsum(sacc)
          svecs[h] = jnp.where(lane == t, s_h, svecs[h])
      jvec = c * G + g * 16 + lane
      wvs = []
      lv_n = []
      for h in range(HPG):
        wv = jnp.where(jvec < nv_v, jnp.exp(svecs[h] * SCALE), 0.0)
        wvs.append(wv)
        lv_n.append(lv[h] + wv)
      for t in range(16):
        tok = g * 16 + t
        for h in range(HPG):
          w_s = _lanesplat(wvs[h], t)
          for j in range(8):
            vx = vbuf[slot, tok, pl.ds(h * D + j * 16, 16)]
            a[h * 8 + j] = a[h * 8 + j] + vx * w_s
      return tuple(lv_n), tuple(a)

    return lax.fori_loop(0, G // 16, group, (lvec, acc))

  lvec, acc = lax.fori_loop(
      0, NCHK, chunk_kv,
      (tuple(jnp.zeros((16,), jnp.float32) for _ in range(HPG)),
       tuple(jnp.zeros((16,), jnp.float32) for _ in range(HPG * 8))))

  for h in range(HPG):
    wc_h = jnp.exp(scur[h])
    l_h = _allsum(lvec[h]) + wc_h
    inv_l = 1.0 / l_h
    for j in range(8):
      vx = vcv[pl.ds(h * D + j * 16, 16)]
      outv[pl.ds(h * D + j * 16, 16)] = (acc[h * 8 + j] + vx * wc_h) * inv_l
  pltpu.sync_copy(outv, out_h.at[i, pl.ds(hg * ROW, ROW)])


def _sc_call(q, k, v, key_cache, value_cache, block_tables, seq_lens):
  kc2 = key_cache.reshape(NUM_BLOCKS * BS * HG, ROW)
  vc2 = value_cache.reshape(NUM_BLOCKS * BS * HG, ROW)
  sck = pl.kernel(
      _sc_body,
      out_type=jax.ShapeDtypeStruct((SC_BATCHES, H * D), jnp.float32),
      mesh=plsc.VectorSubcoreMesh(core_axis_name="c", subcore_axis_name="s",
                                  num_cores=2, num_subcores=16),
      scratch_types=[
          pltpu.VMEM((16,), jnp.int32),
          pltpu.VMEM((MAXB,), jnp.int32),
          pltpu.VMEM((MAXB * BS,), jnp.int32),
          pltpu.VMEM((MAXB * BS,), jnp.int32),
          pltpu.VMEM((2, G, ROW), jnp.float32),
          pltpu.VMEM((2, G, ROW), jnp.float32),
          pltpu.VMEM((2, G, D), jnp.float32),
          pltpu.VMEM((16, D), jnp.float32),
          pltpu.VMEM((ROW,), jnp.float32),
          pltpu.VMEM((ROW,), jnp.float32),
          pltpu.VMEM((ROW,), jnp.float32),
          pltpu.VMEM((ROW,), jnp.float32),
          pltpu.SemaphoreType.DMA((2,)),
          pltpu.SemaphoreType.DMA((2,)),
          pltpu.SemaphoreType.DMA((2,)),
          pltpu.SemaphoreType.DMA,
      ],
  )
  return sck(q, k, v, kc2, vc2, block_tables, seq_lens, jnp.asarray(_PTAB))


CH = 8
T = CH * BS
NCH = (MAXB * BS) // T

_DOT_MINOR = (((1,), (0,)), ((), ()))

_inv = (ROPE_BASE ** (-np.arange(HALF) / HALF))[None, :]


def _ctab(pos_col):
  ang = pos_col[:, None, :] * _inv[None]
  n = ang.shape[0]
  c = np.broadcast_to(np.cos(ang), (n, H, HALF)).astype(np.float32).copy()
  s = np.broadcast_to(np.sin(ang), (n, H, HALF)).astype(np.float32).copy()
  return c, s

_BASE_C, _BASE_S = _ctab(np.arange(T)[:, None].astype(np.float64))
_DELTA_C, _DELTA_S = _ctab((np.arange(NCH) * T)[:, None].astype(np.float64))
_SHIFT_C, _SHIFT_S = _ctab((BS - 1 - np.arange(BS))[:, None].astype(np.float64))


def _attn_body(bt_ref, sl_ref, q_ref, k_ref, v_ref,
               bC_ref, bS_ref, dC_ref, dS_ref, hC_ref, hS_ref,
               kc_ref, vc_ref, o_ref, kbuf, vbuf, ksem, vsem):
  i = pl.program_id(0)

  s = 257 + sl_ref[i] % (2048 - 257)
  num_past = s - 1
  rem = num_past % BS
  within = num_past < CTX
  full = jnp.where(within, num_past // BS, (CTX // BS) - 1)
  n_valid = full * BS + rem
  nblocks = (n_valid + BS - 1) // BS
  nchunks = (nblocks + CH - 1) // CH

  def copies(c, slot):
    out = []
    for b in range(CH):
      safe = jnp.minimum(c * CH + b, nblocks - 1)
      bt = bt_ref[i, safe]
      out.append(pltpu.make_async_copy(
          kc_ref.at[bt], kbuf.at[slot, pl.ds(b * BS, BS)], ksem.at[slot]))
      out.append(pltpu.make_async_copy(
          vc_ref.at[bt], vbuf.at[slot, pl.ds(b * BS, BS)], vsem.at[slot]))
    return out

  def issue(c, slot):
    for cp in copies(c, slot):
      cp.start()

  def wait(c, slot):
    for cp in copies(c, slot):
      cp.wait()

  issue(0, 0)

  baseC = bC_ref[...]
  baseS = bS_ref[...]
  shc = hC_ref[rem]
  shs = hS_ref[rem]
  sbC = baseC * shc - baseS * shs
  sbS = baseS * shc + baseC * shs
  beyond = jnp.logical_not(within)
  jt3 = jax.lax.broadcasted_iota(jnp.int32, (T, H, HALF), 0)
  sink0 = jnp.logical_and(beyond, jt3 >= BS)
  b0C = jnp.where(sink0, sbC, baseC)
  b0S = jnp.where(sink0, sbS, baseS)
  bNC = jnp.where(beyond, sbC, baseC)
  bNS = jnp.where(beyond, sbS, baseS)
  jt1 = jax.lax.broadcasted_iota(jnp.int32, (T, 1, 1), 0)
  ones_half = jnp.ones((HALF, 1), jnp.float32)

  cur_pos = jnp.minimum(num_past, CTX - 1)
  cq = cur_pos // T
  rq = cur_pos % T
  qdc = dC_ref[cq]
  qds = dS_ref[cq]
  qbc = bC_ref[rq]
  qbs = bS_ref[rq]
  qcc = qbc * qdc - qbs * qds
  qss = qbs * qdc + qbc * qds
  def _rot2(x_ref):
    x1 = x_ref[0, :, :HALF]
    x2 = x_ref[0, :, HALF:]
    return x1 * qcc - x2 * qss, x2 * qcc + x1 * qss
  q1, q2 = _rot2(q_ref)
  k1, k2 = _rot2(k_ref)

  def chunk_body(c, carry):
    m, l, acc = carry
    slot = jax.lax.rem(c, 2)

    @pl.when(c + 1 < nchunks)
    def _():
      issue(c + 1, 1 - slot)

    wait(c, slot)

    is0 = c == 0
    bpC = jnp.where(is0, b0C, bNC)
    bpS = jnp.where(is0, b0S, bNS)
    dc = dC_ref[c]
    ds = dS_ref[c]
    PC = bpC * dc - bpS * ds
    PS = bpS * dc + bpC * ds
    mask3 = (c * T + jt1) < n_valid

    X = kbuf[slot]
    x1 = X[..., :HALF]
    x2 = X[..., HALF:]
    r1 = x1 * PC - x2 * PS
    r2 = x2 * PC + x1 * PS
    t3 = r1 * q1 + r2 * q2
    sc = jax.lax.dot_general(t3.reshape(T * H, HALF), ones_half, _DOT_MINOR,
                             preferred_element_type=jnp.float32)
    sc3 = sc.reshape(T, H, 1) * SCALE
    sc3 = jnp.where(mask3, sc3, -1e30)
    m_c = jnp.max(sc3, axis=0, keepdims=True)
    m_new = jnp.maximum(m, m_c)
    alpha = jnp.exp(m - m_new)
    p3 = jnp.exp(sc3 - m_new)
    l_new = alpha * l + jnp.sum(p3, axis=0, keepdims=True)
    W3 = vbuf[slot] * p3
    pv = jnp.sum(W3, axis=0, keepdims=True)
    acc_new = alpha * acc + pv
    return m_new, l_new, acc_new

  m0 = jnp.full((1, H, 1), -1e30, jnp.float32)
  l0 = jnp.zeros((1, H, 1), jnp.float32)
  a0 = jnp.zeros((1, H, D), jnp.float32)
  m, l, acc = jax.lax.fori_loop(0, nchunks, chunk_body, (m0, l0, a0))

  t_cur = q1 * k1 + q2 * k2
  s_cur = jax.lax.dot_general(t_cur, ones_half, _DOT_MINOR,
                              preferred_element_type=jnp.float32)
  s_cur3 = s_cur.reshape(1, H, 1) * SCALE
  m_f = jnp.maximum(m, s_cur3)
  alpha = jnp.exp(m - m_f)
  p_cur = jnp.exp(s_cur3 - m_f)
  l_f = alpha * l + p_cur
  o_ref[...] = (alpha * acc + p_cur * v_ref[...]) / l_f


def _tc_call(q, k, v, key_cache, value_cache, block_tables, seq_lens):
  whole = lambda shape: pl.BlockSpec(shape, lambda i, bt, sl: (0,) * len(shape))
  row = pl.BlockSpec((1, H, D), lambda i, bt, sl: (i, 0, 0))
  grid_spec = pltpu.PrefetchScalarGridSpec(
      num_scalar_prefetch=2,
      grid=(TC_BATCHES,),
      in_specs=[
          row, row, row,
          whole((T, H, HALF)), whole((T, H, HALF)),
          whole((NCH, H, HALF)), whole((NCH, H, HALF)),
          whole((BS, H, HALF)), whole((BS, H, HALF)),
          pl.BlockSpec(memory_space=pl.MemorySpace.ANY),
          pl.BlockSpec(memory_space=pl.MemorySpace.ANY),
      ],
      out_specs=row,
      scratch_shapes=[
          pltpu.VMEM((2, T, H, D), jnp.float32),
          pltpu.VMEM((2, T, H, D), jnp.float32),
          pltpu.SemaphoreType.DMA((2,)),
          pltpu.SemaphoreType.DMA((2,)),
      ],
  )
  out = pl.pallas_call(
      _attn_body,
      grid_spec=grid_spec,
      out_shape=jax.ShapeDtypeStruct((TC_BATCHES, H, D), jnp.float32),
  )(block_tables, seq_lens, q.reshape(B, H, D), k.reshape(B, H, D),
    v.reshape(B, H, D),
    jnp.asarray(_BASE_C), jnp.asarray(_BASE_S),
    jnp.asarray(_DELTA_C), jnp.asarray(_DELTA_S),
    jnp.asarray(_SHIFT_C), jnp.asarray(_SHIFT_S),
    key_cache, value_cache)
  return out.reshape(TC_BATCHES, H * D)


@jax.jit
def kernel(q, k, v, key_cache, value_cache, block_tables, seq_lens, positions):
  del positions
  sc_out = _sc_call(q, k, v, key_cache, value_cache, block_tables, seq_lens)
  tc_out = _tc_call(q, k, v, key_cache, value_cache, block_tables, seq_lens)
  return jnp.concatenate([tc_out, sc_out], axis=0)


# speedup vs baseline: 1.0213x; 1.0213x over previous
import math

import jax
import jax.numpy as jnp
import numpy as np
from jax import lax
from jax.experimental import pallas as pl
from jax.experimental.pallas import tpu as pltpu
from jax.experimental.pallas import tpu_sc as plsc

B = 16
H = 8
D = 128
BS = 16
CTX = 1024
NUM_BLOCKS = 1024
MAXB = 64
ROPE_BASE = 10000.0
HALF = D // 2
SCALE = 1.0 / math.sqrt(D)

HG = 8
HPG = H // HG
ROW = HPG * D
G = 64
NCHK = MAXB * BS // G
NPOS = 2048
TC_BATCHES = 12
SC_BATCHES = B - TC_BATCHES

_invf = ROPE_BASE ** (-np.arange(HALF) / HALF)
_ang = np.arange(NPOS)[:, None] * _invf[None, :]
_PTAB = np.concatenate([np.cos(_ang), np.sin(_ang)], axis=1).astype(np.float32)


_GDN = lax.GatherDimensionNumbers(offset_dims=(), collapsed_slice_dims=(0,),
                                  start_index_map=(0,))


def _iota16():
  return lax.broadcasted_iota(jnp.int32, (16,), 0)


def _dyngather(vec, idxv):
  return lax.gather(vec, idxv[:, None], _GDN, (1,),
                    mode=lax.GatherScatterMode.PROMISE_IN_BOUNDS)


def _allsum(v):
  for sh in (1, 2, 4, 8):
    v = v + _dyngather(v, (_iota16() + sh) & 15)
  return v


def _lanesplat(vec, t):
  return _dyngather(vec, jnp.zeros((16,), jnp.int32) + t)


def _sc_body(q_h, k_h, v_h, kc_h, vc_h, bt_h, sl_h, tab_h, out_h,
             slv, btv, idxr, pidxr, kbuf, vbuf, cbuf, qcs,
             qv, kcv, vcv, outv, ksem, vsem, csem, qsem):
  cid = lax.axis_index("c")
  sid = lax.axis_index("s")
  wid = sid * 2 + cid
  i = wid // HG
  ig = i + TC_BATCHES
  hg = wid % HG
  lane = lax.broadcasted_iota(jnp.int32, (16,), 0)

  pltpu.sync_copy(sl_h, slv)
  pltpu.sync_copy(bt_h.at[ig], btv)
  pltpu.sync_copy(q_h.at[ig, pl.ds(hg * ROW, ROW)], qv)
  pltpu.sync_copy(k_h.at[ig, pl.ds(hg * ROW, ROW)], kcv)
  pltpu.sync_copy(v_h.at[ig, pl.ds(hg * ROW, ROW)], vcv)

  sl_v = _dyngather(slv[...], jnp.full((16,), ig, jnp.int32))
  s_v = 257 + sl_v % (2048 - 257)
  np_v = s_v - 1
  rem_v = np_v % BS
  within_v = np_v < CTX
  nv_v = jnp.where(within_v, np_v, (CTX // BS - 1) * BS + rem_v)
  within_n = (np_v + (lane >> 4)) < CTX
  cur_v = jnp.minimum(np_v, CTX - 1)

  for bb in range(MAXB // 16):
    btq = btv[pl.ds(bb * 16, 16)]
    for b16 in range(16):
      b = bb * 16 + b16
      bt_v = _lanesplat(btq, b16)
      j = b * BS + lane
      rowi = (bt_v * BS + lane) * HG + hg
      posj = jnp.where(within_n, j,
                       jnp.where(j < BS, j, j + BS - 1 - rem_v))
      idxr[pl.ds(b * BS, BS)] = rowi
      pidxr[pl.ds(b * BS, BS)] = posj

  pltpu.async_copy(tab_h.at[cur_v], qcs,
                   qsem).wait()
  q1r, q2r, scur = [], [], []
  for h in range(HPG):
    acc = None
    r1l, r2l = [], []
    for j in range(4):
      c_ = qcs[0, pl.ds(j * 16, 16)]
      s_ = qcs[0, pl.ds(HALF + j * 16, 16)]
      qx1 = qv[pl.ds(h * D + j * 16, 16)]
      qx2 = qv[pl.ds(h * D + HALF + j * 16, 16)]
      kx1 = kcv[pl.ds(h * D + j * 16, 16)]
      kx2 = kcv[pl.ds(h * D + HALF + j * 16, 16)]
      q1 = qx1 * c_ - qx2 * s_
      q2 = qx2 * c_ + qx1 * s_
      k1 = kx1 * c_ - kx2 * s_
      k2 = kx2 * c_ + kx1 * s_
      r1l.append(q1)
      r2l.append(q2)
      t_ = q1 * k1 + q2 * k2
      acc = t_ if acc is None else acc + t_
    q1r.append(r1l)
    q2r.append(r2l)
    scur.append(_allsum(acc) * SCALE)

  def issue(c):
    slot = c % 2
    pltpu.make_async_copy(kc_h.at[idxr.at[pl.ds(c * G, G)]],
                          kbuf.at[slot], ksem.at[slot]).start()
    pltpu.make_async_copy(vc_h.at[idxr.at[pl.ds(c * G, G)]],
                          vbuf.at[slot], vsem.at[slot]).start()
    pltpu.make_async_copy(tab_h.at[pidxr.at[pl.ds(c * G, G)]],
                          cbuf.at[slot], csem.at[slot]).start()

  def wait(c):
    slot = c % 2
    pltpu.make_async_copy(kc_h.at[idxr.at[pl.ds(c * G, G)]],
                          kbuf.at[slot], ksem.at[slot]).wait()
    pltpu.make_async_copy(vc_h.at[idxr.at[pl.ds(c * G, G)]],
                          vbuf.at[slot], vsem.at[slot]).wait()
    pltpu.make_async_copy(tab_h.at[pidxr.at[pl.ds(c * G, G)]],
                          cbuf.at[slot], csem.at[slot]).wait()

  issue(0)

  def chunk_kv(c, carry):
    lvec, acc = carry

    @pl.when(c + 1 < NCHK)
    def _():
      issue(c + 1)
    wait(c)
    slot = c % 2

    def group(g, carry_g):
      lv, a = carry_g
      a = list(a)
      svecs = [jnp.zeros((16,), jnp.float32) for _ in range(HPG)]
      for t in range(16):
        tok = g * 16 + t
        cvs = [cbuf[slot, tok, pl.ds(j * 16, 16)] for j in range(4)]
        svs = [cbuf[slot, tok, pl.ds(HALF + j * 16, 16)] for j in range(4)]
        for h in range(HPG):
          sacc = None
          for j in range(4):
            x1 = kbuf[slot, tok, pl.ds(h * D + j * 16, 16)]
            x2 = kbuf[slot, tok, pl.ds(h * D + HALF + j * 16, 16)]
            r1 = x1 * cvs[j] - x2 * svs[j]
            r2 = x2 * cvs[j] + x1 * svs[j]
            t_ = r1 * q1r[h][j] + r2 * q2r[h][j]
            sacc = t_ if sacc is None else sacc + t_
          s_h = _allsum(sacc)
          svecs[h] = jnp.where(lane == t, s_h, svecs[h])
      jvec = c * G + g * 16 + lane
      wvs = []
      lv_n = []
      for h in range(HPG):
        wv = jnp.where(jvec < nv_v, jnp.exp(svecs[h] * SCALE), 0.0)
        wvs.append(wv)
        lv_n.append(lv[h] + wv)
      for t in range(16):
        tok = g * 16 + t
        for h in range(HPG):
          w_s = _lanesplat(wvs[h], t)
          for j in range(8):
            vx = vbuf[slot, tok, pl.ds(h * D + j * 16, 16)]
            a[h * 8 + j] = a[h * 8 + j] + vx * w_s
      return tuple(lv_n), tuple(a)

    return lax.fori_loop(0, G // 16, group, (lvec, acc))

  lvec, acc = lax.fori_loop(
      0, NCHK, chunk_kv,
      (tuple(jnp.zeros((16,), jnp.float32) for _ in range(HPG)),
       tuple(jnp.zeros((16,), jnp.float32) for _ in range(HPG * 8))))

  for h in range(HPG):
    wc_h = jnp.exp(scur[h])
    l_h = _allsum(lvec[h]) + wc_h
    inv_l = 1.0 / l_h
    for j in range(8):
      vx = vcv[pl.ds(h * D + j * 16, 16)]
      outv[pl.ds(h * D + j * 16, 16)] = (acc[h * 8 + j] + vx * wc_h) * inv_l
  pltpu.sync_copy(outv, out_h.at[i, pl.ds(hg * ROW, ROW)])


def _sc_call(q, k, v, key_cache, value_cache, block_tables, seq_lens):
  kc2 = key_cache.reshape(NUM_BLOCKS * BS * HG, ROW)
  vc2 = value_cache.reshape(NUM_BLOCKS * BS * HG, ROW)
  sck = pl.kernel(
      _sc_body,
      out_type=jax.ShapeDtypeStruct((SC_BATCHES, H * D), jnp.float32),
      mesh=plsc.VectorSubcoreMesh(core_axis_name="c", subcore_axis_name="s",
                                  num_cores=2, num_subcores=16),
      scratch_types=[
          pltpu.VMEM((16,), jnp.int32),
          pltpu.VMEM((MAXB,), jnp.int32),
          pltpu.VMEM((MAXB * BS,), jnp.int32),
          pltpu.VMEM((MAXB * BS,), jnp.int32),
          pltpu.VMEM((2, G, ROW), jnp.float32),
          pltpu.VMEM((2, G, ROW), jnp.float32),
          pltpu.VMEM((2, G, D), jnp.float32),
          pltpu.VMEM((16, D), jnp.float32),
          pltpu.VMEM((ROW,), jnp.float32),
          pltpu.VMEM((ROW,), jnp.float32),
          pltpu.VMEM((ROW,), jnp.float32),
          pltpu.VMEM((ROW,), jnp.float32),
          pltpu.SemaphoreType.DMA((2,)),
          pltpu.SemaphoreType.DMA((2,)),
          pltpu.SemaphoreType.DMA((2,)),
          pltpu.SemaphoreType.DMA,
      ],
  )
  return sck(q, k, v, kc2, vc2, block_tables, seq_lens, jnp.asarray(_PTAB))


CH = 8
T = CH * BS
NCH = (MAXB * BS) // T

_DOT_MINOR = (((1,), (0,)), ((), ()))

_inv = (ROPE_BASE ** (-np.arange(HALF) / HALF))[None, :]


def _ctab(pos_col):
  ang = pos_col[:, None, :] * _inv[None]
  n = ang.shape[0]
  c = np.broadcast_to(np.cos(ang), (n, H, HALF)).astype(np.float32).copy()
  s = np.broadcast_to(np.sin(ang), (n, H, HALF)).astype(np.float32).copy()
  return c, s

_BASE_C, _BASE_S = _ctab(np.arange(T)[:, None].astype(np.float64))
_DELTA_C, _DELTA_S = _ctab((np.arange(NCH) * T)[:, None].astype(np.float64))
_SHIFT_C, _SHIFT_S = _ctab((BS - 1 - np.arange(BS))[:, None].astype(np.float64))


def _attn_body(bt_ref, sl_ref, q_ref, k_ref, v_ref,
               bC_ref, bS_ref, dC_ref, dS_ref, hC_ref, hS_ref,
               kc_ref, vc_ref, o_ref, kbuf, vbuf, ksem, vsem):
  i = pl.program_id(0)

  s = 257 + sl_ref[i] % (2048 - 257)
  num_past = s - 1
  rem = num_past % BS
  within = num_past < CTX
  full = jnp.where(within, num_past // BS, (CTX // BS) - 1)
  n_valid = full * BS + rem
  nblocks = (n_valid + BS - 1) // BS
  nchunks = (nblocks + CH - 1) // CH

  def copies(c, slot):
    out = []
    for b in range(CH):
      safe = jnp.minimum(c * CH + b, nblocks - 1)
      bt = bt_ref[i, safe]
      out.append(pltpu.make_async_copy(
          kc_ref.at[bt], kbuf.at[slot, pl.ds(b * BS, BS)], ksem.at[slot]))
      out.append(pltpu.make_async_copy(
          vc_ref.at[bt], vbuf.at[slot, pl.ds(b * BS, BS)], vsem.at[slot]))
    return out

  def issue(c, slot):
    for cp in copies(c, slot):
      cp.start()

  def wait(c, slot):
    for cp in copies(c, slot):
      cp.wait()

  issue(0, 0)

  baseC = bC_ref[...]
  baseS = bS_ref[...]
  shc = hC_ref[rem]
  shs = hS_ref[rem]
  sbC = baseC * shc - baseS * shs
  sbS = baseS * shc + baseC * shs
  beyond = jnp.logical_not(within)
  jt3 = jax.lax.broadcasted_iota(jnp.int32, (T, H, HALF), 0)
  sink0 = jnp.logical_and(beyond, jt3 >= BS)
  b0C = jnp.where(sink0, sbC, baseC)
  b0S = jnp.where(sink0, sbS, baseS)
  bNC = jnp.where(beyond, sbC, baseC)
  bNS = jnp.where(beyond, sbS, baseS)
  jt1 = jax.lax.broadcasted_iota(jnp.int32, (T, 1, 1), 0)
  ones_half = jnp.ones((HALF, 1), jnp.float32)

  cur_pos = jnp.minimum(num_past, CTX - 1)
  cq = cur_pos // T
  rq = cur_pos % T
  qdc = dC_ref[cq]
  qds = dS_ref[cq]
  qbc = bC_ref[rq]
  qbs = bS_ref[rq]
  qcc = qbc * qdc - qbs * qds
  qss = qbs * qdc + qbc * qds
  def _rot2(x_ref):
    x1 = x_ref[0, :, :HALF]
    x2 = x_ref[0, :, HALF:]
    return x1 * qcc - x2 * qss, x2 * qcc + x1 * qss
  q1, q2 = _rot2(q_ref)
  k1, k2 = _rot2(k_ref)
  q1 = q1 * SCALE
  q2 = q2 * SCALE

  def chunk_body(c, carry):
    m, l, acc = carry
    slot = jax.lax.rem(c, 2)

    @pl.when(c + 1 < nchunks)
    def _():
      issue(c + 1, 1 - slot)

    wait(c, slot)

    is0 = c == 0
    dc = dC_ref[c]
    ds = dS_ref[c]
    PC = jnp.where(is0, b0C, bNC * dc - bNS * ds)
    PS = jnp.where(is0, b0S, bNS * dc + bNC * ds)
    mask3 = (c * T + jt1) < n_valid

    X = kbuf[slot]
    x1 = X[..., :HALF]
    x2 = X[..., HALF:]
    r1 = x1 * PC - x2 * PS
    r2 = x2 * PC + x1 * PS
    t3 = r1 * q1 + r2 * q2
    sc = jax.lax.dot_general(t3.reshape(T * H, HALF), ones_half, _DOT_MINOR,
                             preferred_element_type=jnp.float32)
    sc3 = sc.reshape(T, H, 1)
    sc3 = jnp.where(mask3, sc3, -1e30)
    m_c = jnp.max(sc3, axis=0, keepdims=True)
    m_new = jnp.maximum(m, m_c)
    alpha = jnp.exp(m - m_new)
    p3 = jnp.exp(sc3 - m_new)
    l_new = alpha * l + jnp.sum(p3, axis=0, keepdims=True)
    W3 = vbuf[slot] * p3
    pv = jnp.sum(W3, axis=0, keepdims=True)
    acc_new = alpha * acc + pv
    return m_new, l_new, acc_new

  m0 = jnp.full((1, H, 1), -1e30, jnp.float32)
  l0 = jnp.zeros((1, H, 1), jnp.float32)
  a0 = jnp.zeros((1, H, D), jnp.float32)
  m, l, acc = jax.lax.fori_loop(0, nchunks, chunk_body, (m0, l0, a0))

  t_cur = q1 * k1 + q2 * k2
  s_cur = jax.lax.dot_general(t_cur, ones_half, _DOT_MINOR,
                              preferred_element_type=jnp.float32)
  s_cur3 = s_cur.reshape(1, H, 1)
  m_f = jnp.maximum(m, s_cur3)
  alpha = jnp.exp(m - m_f)
  p_cur = jnp.exp(s_cur3 - m_f)
  l_f = alpha * l + p_cur
  o_ref[...] = (alpha * acc + p_cur * v_ref[...]) / l_f


def _tc_call(q, k, v, key_cache, value_cache, block_tables, seq_lens):
  whole = lambda shape: pl.BlockSpec(shape, lambda i, bt, sl: (0,) * len(shape))
  row = pl.BlockSpec((1, H, D), lambda i, bt, sl: (i, 0, 0))
  grid_spec = pltpu.PrefetchScalarGridSpec(
      num_scalar_prefetch=2,
      grid=(TC_BATCHES,),
      in_specs=[
          row, row, row,
          whole((T, H, HALF)), whole((T, H, HALF)),
          whole((NCH, H, HALF)), whole((NCH, H, HALF)),
          whole((BS, H, HALF)), whole((BS, H, HALF)),
          pl.BlockSpec(memory_space=pl.MemorySpace.ANY),
          pl.BlockSpec(memory_space=pl.MemorySpace.ANY),
      ],
      out_specs=row,
      scratch_shapes=[
          pltpu.VMEM((2, T, H, D), jnp.float32),
          pltpu.VMEM((2, T, H, D), jnp.float32),
          pltpu.SemaphoreType.DMA((2,)),
          pltpu.SemaphoreType.DMA((2,)),
      ],
  )
  out = pl.pallas_call(
      _attn_body,
      grid_spec=grid_spec,
      out_shape=jax.ShapeDtypeStruct((TC_BATCHES, H, D), jnp.float32),
  )(block_tables, seq_lens, q.reshape(B, H, D), k.reshape(B, H, D),
    v.reshape(B, H, D),
    jnp.asarray(_BASE_C), jnp.asarray(_BASE_S),
    jnp.asarray(_DELTA_C), jnp.asarray(_DELTA_S),
    jnp.asarray(_SHIFT_C), jnp.asarray(_SHIFT_S),
    key_cache, value_cache)
  return out.reshape(TC_BATCHES, H * D)


@jax.jit
def kernel(q, k, v, key_cache, value_cache, block_tables, seq_lens, positions):
  del positions
  sc_out = _sc_call(q, k, v, key_cache, value_cache, block_tables, seq_lens)
  tc_out = _tc_call(q, k, v, key_cache, value_cache, block_tables, seq_lens)
  return jnp.concatenate([tc_out, sc_out], axis=0)


# speedup vs baseline: 1.0578x; 1.0357x over previous
import math

import jax
import jax.numpy as jnp
import numpy as np
from jax import lax
from jax.experimental import pallas as pl
from jax.experimental.pallas import tpu as pltpu
from jax.experimental.pallas import tpu_sc as plsc

B = 16
H = 8
D = 128
BS = 16
CTX = 1024
NUM_BLOCKS = 1024
MAXB = 64
ROPE_BASE = 10000.0
HALF = D // 2
SCALE = 1.0 / math.sqrt(D)

HG = 8
HPG = H // HG
ROW = HPG * D
G = 64
NCHK = MAXB * BS // G
NPOS = 2048
TC_BATCHES = 12
SC_BATCHES = B - TC_BATCHES

_invf = ROPE_BASE ** (-np.arange(HALF) / HALF)
_ang = np.arange(NPOS)[:, None] * _invf[None, :]
_PTAB = np.concatenate([np.cos(_ang), np.sin(_ang)], axis=1).astype(np.float32)


_GDN = lax.GatherDimensionNumbers(offset_dims=(), collapsed_slice_dims=(0,),
                                  start_index_map=(0,))


def _iota16():
  return lax.broadcasted_iota(jnp.int32, (16,), 0)


def _dyngather(vec, idxv):
  return lax.gather(vec, idxv[:, None], _GDN, (1,),
                    mode=lax.GatherScatterMode.PROMISE_IN_BOUNDS)


def _allsum(v):
  for sh in (1, 2, 4, 8):
    v = v + _dyngather(v, (_iota16() + sh) & 15)
  return v


def _lanesplat(vec, t):
  return _dyngather(vec, jnp.zeros((16,), jnp.int32) + t)


def _sc_body(q_h, k_h, v_h, kc_h, vc_h, bt_h, sl_h, tab_h, out_h,
             slv, btv, idxr, pidxr, kbuf, vbuf, cbuf, qcs,
             qv, kcv, vcv, outv, ksem, vsem, csem, qsem):
  cid = lax.axis_index("c")
  sid = lax.axis_index("s")
  wid = sid * 2 + cid
  i = wid // HG
  ig = i + TC_BATCHES
  hg = wid % HG
  lane = lax.broadcasted_iota(jnp.int32, (16,), 0)

  pltpu.sync_copy(sl_h, slv)
  pltpu.sync_copy(bt_h.at[ig], btv)
  pltpu.sync_copy(q_h.at[ig, pl.ds(hg * ROW, ROW)], qv)
  pltpu.sync_copy(k_h.at[ig, pl.ds(hg * ROW, ROW)], kcv)
  pltpu.sync_copy(v_h.at[ig, pl.ds(hg * ROW, ROW)], vcv)

  sl_v = _dyngather(slv[...], jnp.full((16,), ig, jnp.int32))
  s_v = 257 + sl_v % (2048 - 257)
  np_v = s_v - 1
  rem_v = np_v % BS
  within_v = np_v < CTX
  nv_v = jnp.where(within_v, np_v, (CTX // BS - 1) * BS + rem_v)
  within_n = (np_v + (lane >> 4)) < CTX
  cur_v = jnp.minimum(np_v, CTX - 1)

  for bb in range(MAXB // 16):
    btq = btv[pl.ds(bb * 16, 16)]
    for b16 in range(16):
      b = bb * 16 + b16
      bt_v = _lanesplat(btq, b16)
      j = b * BS + lane
      rowi = (bt_v * BS + lane) * HG + hg
      posj = jnp.where(within_n, j,
                       jnp.where(j < BS, j, j + BS - 1 - rem_v))
      idxr[pl.ds(b * BS, BS)] = rowi
      pidxr[pl.ds(b * BS, BS)] = posj

  pltpu.async_copy(tab_h.at[cur_v], qcs,
                   qsem).wait()
  q1r, q2r, scur = [], [], []
  for h in range(HPG):
    acc = None
    r1l, r2l = [], []
    for j in range(4):
      c_ = qcs[0, pl.ds(j * 16, 16)]
      s_ = qcs[0, pl.ds(HALF + j * 16, 16)]
      qx1 = qv[pl.ds(h * D + j * 16, 16)]
      qx2 = qv[pl.ds(h * D + HALF + j * 16, 16)]
      kx1 = kcv[pl.ds(h * D + j * 16, 16)]
      kx2 = kcv[pl.ds(h * D + HALF + j * 16, 16)]
      q1 = qx1 * c_ - qx2 * s_
      q2 = qx2 * c_ + qx1 * s_
      k1 = kx1 * c_ - kx2 * s_
      k2 = kx2 * c_ + kx1 * s_
      r1l.append(q1)
      r2l.append(q2)
      t_ = q1 * k1 + q2 * k2
      acc = t_ if acc is None else acc + t_
    q1r.append(r1l)
    q2r.append(r2l)
    scur.append(_allsum(acc) * SCALE)

  def issue(c):
    slot = c % 2
    pltpu.make_async_copy(kc_h.at[idxr.at[pl.ds(c * G, G)]],
                          kbuf.at[slot], ksem.at[slot]).start()
    pltpu.make_async_copy(vc_h.at[idxr.at[pl.ds(c * G, G)]],
                          vbuf.at[slot], vsem.at[slot]).start()
    pltpu.make_async_copy(tab_h.at[pidxr.at[pl.ds(c * G, G)]],
                          cbuf.at[slot], csem.at[slot]).start()

  def wait(c):
    slot = c % 2
    pltpu.make_async_copy(kc_h.at[idxr.at[pl.ds(c * G, G)]],
                          kbuf.at[slot], ksem.at[slot]).wait()
    pltpu.make_async_copy(vc_h.at[idxr.at[pl.ds(c * G, G)]],
                          vbuf.at[slot], vsem.at[slot]).wait()
    pltpu.make_async_copy(tab_h.at[pidxr.at[pl.ds(c * G, G)]],
                          cbuf.at[slot], csem.at[slot]).wait()

  issue(0)

  def chunk_kv(c, carry):
    lvec, acc = carry

    @pl.when(c + 1 < NCHK)
    def _():
      issue(c + 1)
    wait(c)
    slot = c % 2

    def group(g, carry_g):
      lv, a = carry_g
      a = list(a)
      svecs = [jnp.zeros((16,), jnp.float32) for _ in range(HPG)]
      for t in range(16):
        tok = g * 16 + t
        cvs = [cbuf[slot, tok, pl.ds(j * 16, 16)] for j in range(4)]
        svs = [cbuf[slot, tok, pl.ds(HALF + j * 16, 16)] for j in range(4)]
        for h in range(HPG):
          sacc = None
          for j in range(4):
            x1 = kbuf[slot, tok, pl.ds(h * D + j * 16, 16)]
            x2 = kbuf[slot, tok, pl.ds(h * D + HALF + j * 16, 16)]
            r1 = x1 * cvs[j] - x2 * svs[j]
            r2 = x2 * cvs[j] + x1 * svs[j]
            t_ = r1 * q1r[h][j] + r2 * q2r[h][j]
            sacc = t_ if sacc is None else sacc + t_
          s_h = _allsum(sacc)
          svecs[h] = jnp.where(lane == t, s_h, svecs[h])
      jvec = c * G + g * 16 + lane
      wvs = []
      lv_n = []
      for h in range(HPG):
        wv = jnp.where(jvec < nv_v, jnp.exp(svecs[h] * SCALE), 0.0)
        wvs.append(wv)
        lv_n.append(lv[h] + wv)
      for t in range(16):
        tok = g * 16 + t
        for h in range(HPG):
          w_s = _lanesplat(wvs[h], t)
          for j in range(8):
            vx = vbuf[slot, tok, pl.ds(h * D + j * 16, 16)]
            a[h * 8 + j] = a[h * 8 + j] + vx * w_s
      return tuple(lv_n), tuple(a)

    return lax.fori_loop(0, G // 16, group, (lvec, acc))

  lvec, acc = lax.fori_loop(
      0, NCHK, chunk_kv,
      (tuple(jnp.zeros((16,), jnp.float32) for _ in range(HPG)),
       tuple(jnp.zeros((16,), jnp.float32) for _ in range(HPG * 8))))

  for h in range(HPG):
    wc_h = jnp.exp(scur[h])
    l_h = _allsum(lvec[h]) + wc_h
    inv_l = 1.0 / l_h
    for j in range(8):
      vx = vcv[pl.ds(h * D + j * 16, 16)]
      outv[pl.ds(h * D + j * 16, 16)] = (acc[h * 8 + j] + vx * wc_h) * inv_l
  pltpu.sync_copy(outv, out_h.at[i, pl.ds(hg * ROW, ROW)])


def _sc_call(q, k, v, key_cache, value_cache, block_tables, seq_lens):
  kc2 = key_cache.reshape(NUM_BLOCKS * BS * HG, ROW)
  vc2 = value_cache.reshape(NUM_BLOCKS * BS * HG, ROW)
  sck = pl.kernel(
      _sc_body,
      out_type=jax.ShapeDtypeStruct((SC_BATCHES, H * D), jnp.float32),
      mesh=plsc.VectorSubcoreMesh(core_axis_name="c", subcore_axis_name="s",
                                  num_cores=2, num_subcores=16),
      scratch_types=[
          pltpu.VMEM((16,), jnp.int32),
          pltpu.VMEM((MAXB,), jnp.int32),
          pltpu.VMEM((MAXB * BS,), jnp.int32),
          pltpu.VMEM((MAXB * BS,), jnp.int32),
          pltpu.VMEM((2, G, ROW), jnp.float32),
          pltpu.VMEM((2, G, ROW), jnp.float32),
          pltpu.VMEM((2, G, D), jnp.float32),
          pltpu.VMEM((16, D), jnp.float32),
          pltpu.VMEM((ROW,), jnp.float32),
          pltpu.VMEM((ROW,), jnp.float32),
          pltpu.VMEM((ROW,), jnp.float32),
          pltpu.VMEM((ROW,), jnp.float32),
          pltpu.SemaphoreType.DMA((2,)),
          pltpu.SemaphoreType.DMA((2,)),
          pltpu.SemaphoreType.DMA((2,)),
          pltpu.SemaphoreType.DMA,
      ],
  )
  return sck(q, k, v, kc2, vc2, block_tables, seq_lens, jnp.asarray(_PTAB))


CH = 16
T = CH * BS
NCH = (MAXB * BS) // T

_DOT_MINOR = (((1,), (0,)), ((), ()))

_inv = (ROPE_BASE ** (-np.arange(HALF) / HALF))[None, :]


def _ctab(pos_col):
  ang = pos_col[:, None, :] * _inv[None]
  n = ang.shape[0]
  c = np.broadcast_to(np.cos(ang), (n, H, HALF)).astype(np.float32).copy()
  s = np.broadcast_to(np.sin(ang), (n, H, HALF)).astype(np.float32).copy()
  return c, s

_BASE_C, _BASE_S = _ctab(np.arange(T)[:, None].astype(np.float64))
_DELTA_C, _DELTA_S = _ctab((np.arange(NCH) * T)[:, None].astype(np.float64))
_SHIFT_C, _SHIFT_S = _ctab((BS - 1 - np.arange(BS))[:, None].astype(np.float64))


def _attn_body(bt_ref, sl_ref, q_ref, k_ref, v_ref,
               bC_ref, bS_ref, dC_ref, dS_ref, hC_ref, hS_ref,
               kc_ref, vc_ref, o_ref, kbuf, vbuf, ksem, vsem):
  i = pl.program_id(0)

  s = 257 + sl_ref[i] % (2048 - 257)
  num_past = s - 1
  rem = num_past % BS
  within = num_past < CTX
  full = jnp.where(within, num_past // BS, (CTX // BS) - 1)
  n_valid = full * BS + rem
  nblocks = (n_valid + BS - 1) // BS
  nchunks = (nblocks + CH - 1) // CH

  def copies(c, slot):
    out = []
    for b in range(CH):
      safe = jnp.minimum(c * CH + b, nblocks - 1)
      bt = bt_ref[i, safe]
      out.append(pltpu.make_async_copy(
          kc_ref.at[bt], kbuf.at[slot, pl.ds(b * BS, BS)], ksem.at[slot]))
      out.append(pltpu.make_async_copy(
          vc_ref.at[bt], vbuf.at[slot, pl.ds(b * BS, BS)], vsem.at[slot]))
    return out

  def issue(c, slot):
    for cp in copies(c, slot):
      cp.start()

  def wait(c, slot):
    for cp in copies(c, slot):
      cp.wait()

  issue(0, 0)

  baseC = bC_ref[...]
  baseS = bS_ref[...]
  shc = hC_ref[rem]
  shs = hS_ref[rem]
  sbC = baseC * shc - baseS * shs
  sbS = baseS * shc + baseC * shs
  beyond = jnp.logical_not(within)
  jt3 = jax.lax.broadcasted_iota(jnp.int32, (T, H, HALF), 0)
  sink0 = jnp.logical_and(beyond, jt3 >= BS)
  b0C = jnp.where(sink0, sbC, baseC)
  b0S = jnp.where(sink0, sbS, baseS)
  bNC = jnp.where(beyond, sbC, baseC)
  bNS = jnp.where(beyond, sbS, baseS)
  jt1 = jax.lax.broadcasted_iota(jnp.int32, (T, 1, 1), 0)
  ones_half = jnp.ones((HALF, 1), jnp.float32)

  cur_pos = jnp.minimum(num_past, CTX - 1)
  cq = cur_pos // T
  rq = cur_pos % T
  qdc = dC_ref[cq]
  qds = dS_ref[cq]
  qbc = bC_ref[rq]
  qbs = bS_ref[rq]
  qcc = qbc * qdc - qbs * qds
  qss = qbs * qdc + qbc * qds
  def _rot2(x_ref):
    x1 = x_ref[0, :, :HALF]
    x2 = x_ref[0, :, HALF:]
    return x1 * qcc - x2 * qss, x2 * qcc + x1 * qss
  q1, q2 = _rot2(q_ref)
  k1, k2 = _rot2(k_ref)
  q1 = q1 * SCALE
  q2 = q2 * SCALE

  def chunk_body(c, carry):
    m, l, acc = carry
    slot = jax.lax.rem(c, 2)

    @pl.when(c + 1 < nchunks)
    def _():
      issue(c + 1, 1 - slot)

    wait(c, slot)

    is0 = c == 0
    dc = dC_ref[c]
    ds = dS_ref[c]
    PC = jnp.where(is0, b0C, bNC * dc - bNS * ds)
    PS = jnp.where(is0, b0S, bNS * dc + bNC * ds)
    mask3 = (c * T + jt1) < n_valid

    X = kbuf[slot]
    x1 = X[..., :HALF]
    x2 = X[..., HALF:]
    r1 = x1 * PC - x2 * PS
    r2 = x2 * PC + x1 * PS
    t3 = r1 * q1 + r2 * q2
    sc = jax.lax.dot_general(t3.reshape(T * H, HALF), ones_half, _DOT_MINOR,
                             preferred_element_type=jnp.float32)
    sc3 = sc.reshape(T, H, 1)
    sc3 = jnp.where(mask3, sc3, -1e30)
    m_c = jnp.max(sc3, axis=0, keepdims=True)
    m_new = jnp.maximum(m, m_c)
    alpha = jnp.exp(m - m_new)
    p3 = jnp.exp(sc3 - m_new)
    l_new = alpha * l + jnp.sum(p3, axis=0, keepdims=True)
    W3 = vbuf[slot] * p3
    pv = jnp.sum(W3, axis=0, keepdims=True)
    acc_new = alpha * acc + pv
    return m_new, l_new, acc_new

  m0 = jnp.full((1, H, 1), -1e30, jnp.float32)
  l0 = jnp.zeros((1, H, 1), jnp.float32)
  a0 = jnp.zeros((1, H, D), jnp.float32)
  m, l, acc = jax.lax.fori_loop(0, nchunks, chunk_body, (m0, l0, a0))

  t_cur = q1 * k1 + q2 * k2
  s_cur = jax.lax.dot_general(t_cur, ones_half, _DOT_MINOR,
                              preferred_element_type=jnp.float32)
  s_cur3 = s_cur.reshape(1, H, 1)
  m_f = jnp.maximum(m, s_cur3)
  alpha = jnp.exp(m - m_f)
  p_cur = jnp.exp(s_cur3 - m_f)
  l_f = alpha * l + p_cur
  o_ref[...] = (alpha * acc + p_cur * v_ref[...]) / l_f


def _tc_call(q, k, v, key_cache, value_cache, block_tables, seq_lens):
  whole = lambda shape: pl.BlockSpec(shape, lambda i, bt, sl: (0,) * len(shape))
  row = pl.BlockSpec((1, H, D), lambda i, bt, sl: (i, 0, 0))
  grid_spec = pltpu.PrefetchScalarGridSpec(
      num_scalar_prefetch=2,
      grid=(TC_BATCHES,),
      in_specs=[
          row, row, row,
          whole((T, H, HALF)), whole((T, H, HALF)),
          whole((NCH, H, HALF)), whole((NCH, H, HALF)),
          whole((BS, H, HALF)), whole((BS, H, HALF)),
          pl.BlockSpec(memory_space=pl.MemorySpace.ANY),
          pl.BlockSpec(memory_space=pl.MemorySpace.ANY),
      ],
      out_specs=row,
      scratch_shapes=[
          pltpu.VMEM((2, T, H, D), jnp.float32),
          pltpu.VMEM((2, T, H, D), jnp.float32),
          pltpu.SemaphoreType.DMA((2,)),
          pltpu.SemaphoreType.DMA((2,)),
      ],
  )
  out = pl.pallas_call(
      _attn_body,
      grid_spec=grid_spec,
      out_shape=jax.ShapeDtypeStruct((TC_BATCHES, H, D), jnp.float32),
  )(block_tables, seq_lens, q.reshape(B, H, D), k.reshape(B, H, D),
    v.reshape(B, H, D),
    jnp.asarray(_BASE_C), jnp.asarray(_BASE_S),
    jnp.asarray(_DELTA_C), jnp.asarray(_DELTA_S),
    jnp.asarray(_SHIFT_C), jnp.asarray(_SHIFT_S),
    key_cache, value_cache)
  return out.reshape(TC_BATCHES, H * D)


@jax.jit
def kernel(q, k, v, key_cache, value_cache, block_tables, seq_lens, positions):
  del positions
  sc_out = _sc_call(q, k, v, key_cache, value_cache, block_tables, seq_lens)
  tc_out = _tc_call(q, k, v, key_cache, value_cache, block_tables, seq_lens)
  return jnp.concatenate([tc_out, sc_out], axis=0)
